# Initial kernel scaffold; baseline (speedup 1.0000x reference)
import jax
import jax.numpy as jnp
from jax.experimental import pallas as pl


def kernel(x, W0, W1, W2, W3, W4, W5):
    raise NotImplementedError("write your pallas kernel here")


# speedup vs baseline: 1.3733x; 1.3733x over previous
import functools

import jax
import jax.numpy as jnp
from jax import lax
from jax.experimental import pallas as pl
from jax.experimental.pallas import tpu as pltpu
from jax.experimental.pallas import tpu_sc as plsc

F = 6
LANES = 16
CHUNK = 128


@functools.lru_cache(maxsize=None)
def _build(n, emb, nc, ns):
    nw = nc * ns
    n_full, tail = divmod(n, CHUNK)
    total = n_full + (1 if tail else 0)
    t_per_w = (total + nw - 1) // nw
    g8 = emb // LANES

    mesh = plsc.VectorSubcoreMesh(core_axis_name="c", subcore_axis_name="s",
                                  num_cores=nc, num_subcores=ns)

    @functools.partial(
        pl.kernel,
        out_type=jax.ShapeDtypeStruct((n, emb), jnp.float32),
        mesh=mesh,
        scratch_types=(
            [pltpu.VMEM((F, CHUNK), jnp.int32)]
            + [pltpu.VMEM((CHUNK, emb), jnp.float32) for _ in range(F)]
            + [pltpu.SemaphoreType.DMA]
        ),
    )
    def run(xt, w0, w1, w2, w3, w4, w5, out, idx_v, b0, b1, b2, b3, b4, b5, sem):
        tables = (w0, w1, w2, w3, w4, w5)
        bufs = (b0, b1, b2, b3, b4, b5)
        wid = lax.axis_index("c") * ns + lax.axis_index("s")

        def do_chunk(base, rows):
            for i in range(F):
                pltpu.sync_copy(xt.at[i, pl.ds(base, rows)],
                                idx_v.at[i, pl.ds(0, rows)])
            cps = [
                pltpu.make_async_copy(
                    tables[i].at[idx_v.at[i, pl.ds(0, rows)]],
                    bufs[i].at[pl.ds(0, rows)],
                    sem,
                )
                for i in range(F)
            ]
            for cp in cps:
                cp.start()
            for cp in cps:
                cp.wait()

            def acc_body(r, _):
                for j in range(g8):
                    sl = pl.ds(j * LANES, LANES)
                    s = b0[r, sl]
                    for b in bufs[1:]:
                        s = s + b[r, sl]
                    b0[r, sl] = s
                return 0

            lax.fori_loop(0, rows, acc_body, 0)
            pltpu.sync_copy(b0.at[pl.ds(0, rows)], out.at[pl.ds(base, rows)])

        def outer(t, _):
            cid = wid + t * nw

            @pl.when(cid < n_full)
            def _():
                do_chunk(cid * CHUNK, CHUNK)

            if tail:
                @pl.when(cid == n_full)
                def _():
                    do_chunk(n_full * CHUNK, tail)

            return 0

        lax.fori_loop(0, t_per_w, outer, 0)

    return run


def kernel(x, W0, W1, W2, W3, W4, W5):
    if x.ndim == 1:
        x = x[:, None]
    n = x.shape[0]
    emb = W0.shape[1]
    xt = x.T.astype(jnp.int32)
    try:
        info = plsc.get_sparse_core_info()
        nc, ns = info.num_cores, info.num_subcores
    except Exception:
        nc, ns = 2, 16
    run = _build(n, emb, nc, ns)
    return run(xt, W0, W1, W2, W3, W4, W5)


# speedup vs baseline: 1.7382x; 1.2657x over previous
import functools

import jax
import jax.numpy as jnp
from jax import lax
from jax.experimental import pallas as pl
from jax.experimental.pallas import tpu as pltpu
from jax.experimental.pallas import tpu_sc as plsc

F = 6
LANES = 16
CHUNK = 128


@functools.lru_cache(maxsize=None)
def _build(n, emb, nc, ns):
    nw = nc * ns
    n_full, tail = divmod(n, CHUNK)
    total = n_full + (1 if tail else 0)
    t_per_w = (total + nw - 1) // nw
    g8 = emb // LANES

    mesh = plsc.VectorSubcoreMesh(core_axis_name="c", subcore_axis_name="s",
                                  num_cores=nc, num_subcores=ns)

    @functools.partial(
        pl.kernel,
        out_type=jax.ShapeDtypeStruct((n, emb), jnp.float32),
        mesh=mesh,
        scratch_types=(
            [pltpu.VMEM((F, CHUNK), jnp.int32)]
            + [pltpu.VMEM((CHUNK, emb), jnp.float32) for _ in range(F)]
            + [pltpu.SemaphoreType.DMA]
        ),
    )
    def run(xt, w0, w1, w2, w3, w4, w5, out, idx_v, b0, b1, b2, b3, b4, b5, sem):
        tables = (w0, w1, w2, w3, w4, w5)
        bufs = (b0, b1, b2, b3, b4, b5)
        wid = lax.axis_index("c") * ns + lax.axis_index("s")

        def do_chunk(base, rows):
            for i in range(F):
                pltpu.sync_copy(xt.at[i, pl.ds(base, rows)],
                                idx_v.at[i, pl.ds(0, rows)])
            pltpu.async_copy(
                tables[0].at[idx_v.at[0, pl.ds(0, rows)]],
                b0.at[pl.ds(0, rows)], sem,
            ).wait()
            cps = [
                pltpu.async_copy(
                    tables[i].at[idx_v.at[i, pl.ds(0, rows)]],
                    b0.at[pl.ds(0, rows)], sem, add=True,
                )
                for i in range(1, F)
            ]
            for cp in cps:
                cp.wait()
            pltpu.sync_copy(b0.at[pl.ds(0, rows)], out.at[pl.ds(base, rows)])

        def outer(t, _):
            cid = wid + t * nw

            @pl.when(cid < n_full)
            def _():
                do_chunk(cid * CHUNK, CHUNK)

            if tail:
                @pl.when(cid == n_full)
                def _():
                    do_chunk(n_full * CHUNK, tail)

            return 0

        lax.fori_loop(0, t_per_w, outer, 0)

    return run


def kernel(x, W0, W1, W2, W3, W4, W5):
    if x.ndim == 1:
        x = x[:, None]
    n = x.shape[0]
    emb = W0.shape[1]
    xt = x.T.astype(jnp.int32)
    try:
        info = plsc.get_sparse_core_info()
        nc, ns = info.num_cores, info.num_subcores
    except Exception:
        nc, ns = 2, 16
    run = _build(n, emb, nc, ns)
    return run(xt, W0, W1, W2, W3, W4, W5)


# speedup vs baseline: 2.9158x; 1.6775x over previous
import functools

import jax
import jax.numpy as jnp
from jax import lax
from jax.experimental import pallas as pl
from jax.experimental.pallas import tpu as pltpu
from jax.experimental.pallas import tpu_sc as plsc

F = 6
LANES = 16
CHUNK = 128
NSLOT = 2


@functools.lru_cache(maxsize=None)
def _build(n, emb, nc, ns):
    nw = nc * ns
    n_full, tail = divmod(n, CHUNK)
    total = n_full + (1 if tail else 0)
    t_per_w = (total + nw - 1) // nw
    g8 = emb // LANES
    pad = CHUNK - tail

    mesh = plsc.VectorSubcoreMesh(core_axis_name="c", subcore_axis_name="s",
                                  num_cores=nc, num_subcores=ns)

    @functools.partial(
        pl.kernel,
        out_type=jax.ShapeDtypeStruct((n, emb), jnp.float32),
        mesh=mesh,
        scratch_types=(
            [pltpu.VMEM((F, CHUNK), jnp.int32) for _ in range(NSLOT)]
            + [pltpu.VMEM((CHUNK, emb), jnp.float32) for _ in range(NSLOT)]
            + [pltpu.SemaphoreType.DMA for _ in range(3 * NSLOT)]
        ),
    )
    def run(xt, w0, w1, w2, w3, w4, w5, out,
            idx0, idx1, acc0, acc1, si0, si1, sa0, sa1, sw0, sw1):
        tables = (w0, w1, w2, w3, w4, w5)
        idx = (idx0, idx1)
        acc = (acc0, acc1)
        sem_idx = (si0, si1)
        sem_add = (sa0, sa1)
        sem_wb = (sw0, sw1)
        wid = lax.axis_index("c") * ns + lax.axis_index("s")
        zvec = jnp.zeros((LANES,), jnp.float32)


        def idx_copies(s, cid, start):
            base = cid * CHUNK
            for i in range(F):
                d = pltpu.make_async_copy(xt.at[i, pl.ds(base, CHUNK)],
                                          idx[s].at[i], sem_idx[s])
                d.start() if start else d.wait()

        def idx_copies_tail(s, start):
            base = n_full * CHUNK
            for i in range(F):
                d = pltpu.make_async_copy(xt.at[i, pl.ds(base, tail)],
                                          idx[s].at[i, pl.ds(0, tail)],
                                          sem_idx[s])
                d.start() if start else d.wait()
                d = pltpu.make_async_copy(xt.at[i, pl.ds(0, pad)],
                                          idx[s].at[i, pl.ds(tail, pad)],
                                          sem_idx[s])
                d.start() if start else d.wait()

        def stage_idx(k, s, start):
            cid = wid + k * nw
            ok = k >= 0 if isinstance(k, int) else True

            if ok:
                @pl.when(jnp.logical_and(k >= 0, cid < n_full))
                def _():
                    idx_copies(s, cid, start)

                if tail:
                    @pl.when(jnp.logical_and(k >= 0, cid == n_full))
                    def _():
                        idx_copies_tail(s, start)

        def stage_zero(k, s):
            @pl.when(jnp.logical_and(k >= 0, wid + k * nw < total))
            def _():
                def body(r, _):
                    for j in range(g8):
                        acc[s][r, pl.ds(j * LANES, LANES)] = zvec
                    return 0
                lax.fori_loop(0, CHUNK, body, 0)

        def stage_adds(k, s, start):
            @pl.when(jnp.logical_and(k >= 0, wid + k * nw < total))
            def _():
                for i in range(F):
                    d = pltpu.make_async_copy(tables[i].at[idx[s].at[i]],
                                              acc[s], sem_add[s])
                    d.start(add=True) if start else d.wait()

        def stage_wb(k, s, start):
            cid = wid + k * nw

            @pl.when(jnp.logical_and(k >= 0, cid < n_full))
            def _():
                d = pltpu.make_async_copy(acc[s],
                                          out.at[pl.ds(cid * CHUNK, CHUNK)],
                                          sem_wb[s])
                d.start() if start else d.wait()

            if tail:
                @pl.when(jnp.logical_and(k >= 0, cid == n_full))
                def _():
                    d = pltpu.make_async_copy(
                        acc[s].at[pl.ds(0, tail)],
                        out.at[pl.ds(n_full * CHUNK, tail)], sem_wb[s])
                    d.start() if start else d.wait()

        stage_idx(0, 0, start=True)

        def body(j, _):
            t = j * NSLOT
            for ph in range(NSLOT):
                k = t + ph
                s = ph
                stage_wb(k - NSLOT, s, start=False)
                stage_zero(k, s)
                stage_idx(k, s, start=False)
                stage_adds(k, s, start=True)
                prev_s = (ph - 1) % NSLOT
                stage_adds(k - 1, prev_s, start=False)
                stage_wb(k - 1, prev_s, start=True)
                stage_idx(k + 1, (ph + 1) % NSLOT, start=True)
            return 0

        n_beats = t_per_w + NSLOT
        lax.fori_loop(0, (n_beats + NSLOT - 1) // NSLOT, body, 0)

    return run


def kernel(x, W0, W1, W2, W3, W4, W5):
    if x.ndim == 1:
        x = x[:, None]
    n = x.shape[0]
    emb = W0.shape[1]
    xt = x.T.astype(jnp.int32)
    try:
        info = plsc.get_sparse_core_info()
        nc, ns = info.num_cores, info.num_subcores
    except Exception:
        nc, ns = 2, 16
    run = _build(n, emb, nc, ns)
    return run(xt, W0, W1, W2, W3, W4, W5)


# speedup vs baseline: 2.9810x; 1.0223x over previous
import functools

import jax
import jax.numpy as jnp
from jax import lax
from jax.experimental import pallas as pl
from jax.experimental.pallas import tpu as pltpu
from jax.experimental.pallas import tpu_sc as plsc

F = 6
LANES = 16
CHUNK = 128
NSLOT = 3


@functools.lru_cache(maxsize=None)
def _build(n, emb, nc, ns):
    nw = nc * ns
    n_full, tail = divmod(n, CHUNK)
    total = n_full + (1 if tail else 0)
    t_per_w = (total + nw - 1) // nw
    g8 = emb // LANES
    pad = CHUNK - tail
    lag = NSLOT - 1

    mesh = plsc.VectorSubcoreMesh(core_axis_name="c", subcore_axis_name="s",
                                  num_cores=nc, num_subcores=ns)

    @functools.partial(
        pl.kernel,
        out_type=jax.ShapeDtypeStruct((n, emb), jnp.float32),
        mesh=mesh,
        scratch_types=(
            [pltpu.VMEM((F, CHUNK), jnp.int32) for _ in range(NSLOT)]
            + [pltpu.VMEM((CHUNK, emb), jnp.float32) for _ in range(NSLOT)]
            + [pltpu.SemaphoreType.DMA for _ in range(3 * NSLOT)]
        ),
    )
    def run(xt, w0, w1, w2, w3, w4, w5, out, *scratch):
        idx = scratch[0:NSLOT]
        acc = scratch[NSLOT:2 * NSLOT]
        sem_idx = scratch[2 * NSLOT:3 * NSLOT]
        sem_add = scratch[3 * NSLOT:4 * NSLOT]
        sem_wb = scratch[4 * NSLOT:5 * NSLOT]
        tables = (w0, w1, w2, w3, w4, w5)
        wid = lax.axis_index("c") * ns + lax.axis_index("s")
        zvec = jnp.zeros((LANES,), jnp.float32)


        def stage_idx(k, s, start):
            cid = wid + k * nw

            @pl.when(jnp.logical_and(k >= 0, cid < n_full))
            def _():
                d = pltpu.make_async_copy(xt.at[:, pl.ds(cid * CHUNK, CHUNK)],
                                          idx[s], sem_idx[s])
                d.start() if start else d.wait()

            if tail:
                @pl.when(jnp.logical_and(k >= 0, cid == n_full))
                def _():
                    for i in range(F):
                        d = pltpu.make_async_copy(
                            xt.at[i, pl.ds(n_full * CHUNK, tail)],
                            idx[s].at[i, pl.ds(0, tail)], sem_idx[s])
                        d.start() if start else d.wait()
                        d = pltpu.make_async_copy(xt.at[i, pl.ds(0, pad)],
                                                  idx[s].at[i, pl.ds(tail, pad)],
                                                  sem_idx[s])
                        d.start() if start else d.wait()

        def stage_zero(k, s):
            @pl.when(jnp.logical_and(k >= 0, wid + k * nw < total))
            def _():
                def body(r, _):
                    for j in range(g8):
                        acc[s][r, pl.ds(j * LANES, LANES)] = zvec
                    return 0
                lax.fori_loop(0, CHUNK, body, 0)

        def stage_adds(k, s, start):
            @pl.when(jnp.logical_and(k >= 0, wid + k * nw < total))
            def _():
                for i in range(F):
                    d = pltpu.make_async_copy(tables[i].at[idx[s].at[i]],
                                              acc[s], sem_add[s])
                    d.start(add=True) if start else d.wait()

        def stage_wb(k, s, start):
            cid = wid + k * nw

            @pl.when(jnp.logical_and(k >= 0, cid < n_full))
            def _():
                d = pltpu.make_async_copy(acc[s],
                                          out.at[pl.ds(cid * CHUNK, CHUNK)],
                                          sem_wb[s])
                d.start() if start else d.wait()

            if tail:
                @pl.when(jnp.logical_and(k >= 0, cid == n_full))
                def _():
                    d = pltpu.make_async_copy(
                        acc[s].at[pl.ds(0, tail)],
                        out.at[pl.ds(n_full * CHUNK, tail)], sem_wb[s])
                    d.start() if start else d.wait()

        stage_idx(0, 0, start=True)

        def body(j, _):
            t = j * NSLOT
            for ph in range(NSLOT):
                k = t + ph
                s = ph
                stage_wb(k - NSLOT, s, start=False)
                stage_zero(k, s)
                stage_idx(k, s, start=False)
                stage_adds(k, s, start=True)
                ds_ = (ph - lag) % NSLOT
                stage_adds(k - lag, ds_, start=False)
                stage_wb(k - lag, ds_, start=True)
                stage_idx(k + 1, (ph + 1) % NSLOT, start=True)
            return 0

        n_beats = t_per_w + NSLOT
        lax.fori_loop(0, (n_beats + NSLOT - 1) // NSLOT, body, 0)

    return run


def kernel(x, W0, W1, W2, W3, W4, W5):
    if x.ndim == 1:
        x = x[:, None]
    n = x.shape[0]
    emb = W0.shape[1]
    xt = x.T.astype(jnp.int32)
    try:
        info = plsc.get_sparse_core_info()
        nc, ns = info.num_cores, info.num_subcores
    except Exception:
        nc, ns = 2, 16
    run = _build(n, emb, nc, ns)
    return run(xt, W0, W1, W2, W3, W4, W5)
